# Initial kernel scaffold; baseline (speedup 1.0000x reference)
import jax
import jax.numpy as jnp
from jax.experimental import pallas as pl


def kernel(x, codebooks):
    raise NotImplementedError("write your pallas kernel here")


# speedup vs baseline: 1.1021x; 1.1021x over previous
import jax
import jax.numpy as jnp
from jax import lax
from jax.experimental import pallas as pl
from jax.experimental.pallas import tpu as pltpu
from jax.experimental.pallas import tpu_sc as plsc

NQ = 4
K = 8192
D = 256
M = 8192

BM = 512
BN = 2048
NB_M = M // BM
NB_N = K // BN

NC = 2
NS = 16
NW = NC * NS
TPW = M // NW


def _assign_body(r_ref, f2_ref, cn_ref, cb_ref, codes_ref, min_s, arg_s):
    ni = pl.program_id(1)
    r = r_ref[...]
    c = cb_ref[...]
    cn = cn_ref[...]
    f2 = f2_ref[...]
    m = lax.dot_general(r.astype(jnp.bfloat16), c.astype(jnp.bfloat16),
                        (((1,), (1,)), ((), ())),
                        preferred_element_type=jnp.float32)
    dist = (f2 + cn) - 2.0 * m
    bmin = jnp.min(dist, axis=1, keepdims=True)
    iota = lax.broadcasted_iota(jnp.int32, (BM, BN), 1) + ni * BN
    barg = jnp.min(jnp.where(dist == bmin, iota, jnp.int32(2**31 - 1)),
                   axis=1, keepdims=True)
    bmin_r = bmin.astype(jnp.bfloat16).astype(jnp.float32)

    @pl.when(ni == 0)
    def _init():
        min_s[...] = bmin_r
        arg_s[...] = barg

    @pl.when(ni > 0)
    def _update():
        better = bmin < min_s[...]
        arg_s[...] = jnp.where(better, barg, arg_s[...])
        min_s[...] = jnp.where(better, bmin_r, min_s[...])

    @pl.when(ni == NB_N - 1)
    def _write():
        codes_ref[...] = arg_s[...]


def _assign(resid_flat, f2, cn, codebook):
    return pl.pallas_call(
        _assign_body,
        grid=(NB_M, NB_N),
        in_specs=[
            pl.BlockSpec((BM, D), lambda mi, ni: (mi, 0)),
            pl.BlockSpec((BM, 1), lambda mi, ni: (mi, 0)),
            pl.BlockSpec((1, BN), lambda mi, ni: (0, ni)),
            pl.BlockSpec((BN, D), lambda mi, ni: (ni, 0)),
        ],
        out_specs=pl.BlockSpec((BM, 1), lambda mi, ni: (mi, 0)),
        out_shape=jax.ShapeDtypeStruct((M, 1), jnp.int32),
        scratch_shapes=[
            pltpu.VMEM((BM, 1), jnp.float32),
            pltpu.VMEM((BM, 1), jnp.int32),
        ],
    )(resid_flat, f2, cn, codebook)


def _sc_body(cb_hbm, codes_hbm, out_hbm, counts_hbm,
             idx_v, rows_v, ones_v, zeros_v, cnt_v, shared_cnt, sem):
    cid = lax.axis_index("c")
    sid = lax.axis_index("s")
    base = (sid * NC + cid) * TPW

    pltpu.sync_copy(codes_hbm.at[pl.ds(base, TPW)], idx_v)
    gather = pltpu.async_copy(cb_hbm.at[idx_v], rows_v, sem)

    def _fill(i, _):
        ones_v[pl.ds(i * 16, 16)] = jnp.full((16,), 1.0, jnp.float32)
        return _
    lax.fori_loop(0, TPW // 16, _fill, 0)

    def _zfill(i, _):
        zeros_v[pl.ds(i * 16, 16)] = jnp.zeros((16,), jnp.float32)
        return _
    lax.fori_loop(0, K // NS // 16, _zfill, 0)

    pltpu.sync_copy(zeros_v, shared_cnt.at[pl.ds(sid * (K // NS), K // NS)])
    plsc.subcore_barrier()
    pltpu.sync_copy(ones_v, shared_cnt.at[idx_v], add=True)
    plsc.subcore_barrier()

    gather.wait()
    pltpu.sync_copy(rows_v, out_hbm.at[pl.ds(base, TPW)])

    @pl.when(sid == 0)
    def _publish():
        pltpu.sync_copy(shared_cnt, cnt_v)
        pltpu.sync_copy(cnt_v, counts_hbm.at[cid])


def _gather_hist(codebook, codes_flat):
    mesh = plsc.VectorSubcoreMesh(core_axis_name="c", subcore_axis_name="s")
    fn = pl.kernel(
        _sc_body,
        mesh=mesh,
        out_type=[
            jax.ShapeDtypeStruct((M, D), jnp.float32),
            jax.ShapeDtypeStruct((NC, K), jnp.float32),
        ],
        scratch_types=[
            pltpu.VMEM((TPW,), jnp.int32),
            pltpu.VMEM((TPW, D), jnp.float32),
            pltpu.VMEM((TPW,), jnp.float32),
            pltpu.VMEM((K // NS,), jnp.float32),
            pltpu.VMEM((K,), jnp.float32),
            pltpu.VMEM_SHARED((K,), jnp.float32),
            pltpu.SemaphoreType.DMA,
        ],
    )
    return fn(codebook, codes_flat)


def _resid_body(r_ref, q_ref, acc_ref, r_out, acc_out, qst_out, loss_out, loss_s):
    mi = pl.program_id(0)
    r = r_ref[...]
    q = q_ref[...]
    d = q - r
    q_st = r + d
    r_out[...] = r - q_st
    acc_out[...] = acc_ref[...] + q_st
    qst_out[...] = q_st
    part = jnp.full((1, 1), jnp.sum(d * d), jnp.float32)

    @pl.when(mi == 0)
    def _init():
        loss_s[...] = part

    @pl.when(mi > 0)
    def _acc():
        loss_s[...] = loss_s[...] + part

    @pl.when(mi == NB_M - 1)
    def _write():
        loss_out[...] = loss_s[...]


def _resid_update(r, q, acc):
    return pl.pallas_call(
        _resid_body,
        grid=(NB_M,),
        in_specs=[
            pl.BlockSpec((BM, D), lambda mi: (mi, 0)),
            pl.BlockSpec((BM, D), lambda mi: (mi, 0)),
            pl.BlockSpec((BM, D), lambda mi: (mi, 0)),
        ],
        out_specs=[
            pl.BlockSpec((BM, D), lambda mi: (mi, 0)),
            pl.BlockSpec((BM, D), lambda mi: (mi, 0)),
            pl.BlockSpec((BM, D), lambda mi: (mi, 0)),
            pl.BlockSpec((1, 1), lambda mi: (0, 0)),
        ],
        out_shape=[
            jax.ShapeDtypeStruct((M, D), jnp.float32),
            jax.ShapeDtypeStruct((M, D), jnp.float32),
            jax.ShapeDtypeStruct((M, D), jnp.float32),
            jax.ShapeDtypeStruct((1, 1), jnp.float32),
        ],
        scratch_shapes=[pltpu.VMEM((1, 1), jnp.float32)],
    )(r, q, acc)


def _final_body(counts_ref, loss_ref, perp_out, total_out):
    counts = counts_ref[...]
    c = counts[:, 0, :] + counts[:, 1, :]
    p = c * jnp.float32(1.0 / M)
    ent = -jnp.sum(p * jnp.log(p + 1e-10), axis=1, keepdims=True)
    perp_out[...] = jnp.exp(ent)
    total_out[...] = jnp.sum(loss_ref[...], axis=0, keepdims=True) * jnp.float32(
        0.25 / (M * D) / NQ)


def _finalize(counts, loss_parts):
    return pl.pallas_call(
        _final_body,
        out_shape=[
            jax.ShapeDtypeStruct((NQ, 1), jnp.float32),
            jax.ShapeDtypeStruct((1, 1), jnp.float32),
        ],
    )(counts, loss_parts)


def kernel(x, codebooks):
    xf = x.reshape(M, D)
    resid = xf
    acc = jnp.zeros((M, D), jnp.float32)
    cn_list = [jnp.sum(codebooks[i] * codebooks[i], axis=1) for i in range(NQ)]
    q_list = []
    counts_list = []
    loss_list = []
    for i in range(NQ):
        r4 = resid.reshape(x.shape)
        f2 = jnp.sum(r4 * r4, axis=3).reshape(M, 1)
        codes = _assign(resid, f2, cn_list[i].reshape(1, K), codebooks[i])
        q, counts = _gather_hist(codebooks[i], codes.reshape(M))
        resid, acc, q_st, loss_part = _resid_update(resid, q, acc)
        q_list.append(q_st.reshape(x.shape))
        counts_list.append(counts)
        loss_list.append(loss_part)
    perp, total = _finalize(jnp.stack(counts_list), jnp.concatenate(loss_list, 0))
    quantized = acc.reshape(x.shape)
    return (quantized, total.reshape(()), jnp.stack(q_list, axis=0),
            perp.reshape(NQ))
